# baseline (speedup 1.0000x reference)
import functools

import jax
import jax.numpy as jnp
from jax import lax
from jax.experimental import pallas as pl
from jax.experimental.pallas import tpu as pltpu
from jax.experimental.pallas import tpu_sc as plsc

NU = 5000
NI = 5000
NN = NU + NI
D = 128
NACC = 10112
RPT = NACC // 16
E2 = 320000
CHUNK = 128
NCH0 = 120
NCH1 = 40
SCH = 8
EPAD = 16 * CHUNK * (NCH0 + NCH1)
E_SC0 = 16 * CHUNK * NCH0

_mesh = plsc.VectorSubcoreMesh(core_axis_name="c", subcore_axis_name="s")


def _deg_body(dst_hbm, out_hbm, idx_v, deg_v, sem):
    cid = lax.axis_index("c")
    sid = lax.axis_index("s")

    zeros16 = jnp.zeros((16,), jnp.float32)

    def zero_body(i, _):
        deg_v[pl.ds(i * 16, 16)] = zeros16
        return ()
    lax.fori_loop(0, NACC // 16, zero_body, ())

    pltpu.sync_copy(dst_hbm.at[cid, sid], idx_v)

    ones16 = jnp.ones((16,), jnp.float32)

    def body(k, _):
        idx16 = idx_v[pl.ds(k * 16, 16)]
        plsc.addupdate_scatter(deg_v, [idx16], ones16)
        return ()
    lax.fori_loop(0, jnp.where(cid == 0, NCH0 * CHUNK // 16,
                               NCH1 * CHUNK // 16), body, ())

    pltpu.sync_copy(deg_v, out_hbm.at[cid, sid])


_deg_kernel = functools.partial(
    pl.kernel,
    out_type=jax.ShapeDtypeStruct((2, 16, NACC), jnp.float32),
    mesh=_mesh,
    compiler_params=pltpu.CompilerParams(needs_layout_passes=False),
    scratch_types=[
        pltpu.VMEM((NCH0 * CHUNK,), jnp.int32),
        pltpu.VMEM((NACC,), jnp.float32),
        pltpu.SemaphoreType.DMA,
    ],
)(_deg_body)


def _spmm_body(y_hbm, src_hbm, dst_hbm, out_hbm, srcv, dstv, rows0, rows1,
               acc_sh, sem0, sem1):
    cid = lax.axis_index("c")
    sid = lax.axis_index("s")

    pltpu.sync_copy(src_hbm.at[cid, sid], srcv)

    zeros16 = jnp.zeros((16,), jnp.float32)

    def zero_body(k, _):
        r = k // (D // 16)
        c = k % (D // 16)
        rows0[r, pl.ds(c * 16, 16)] = zeros16
        return ()
    lax.fori_loop(0, CHUNK * (D // 16), zero_body, ())

    def zcopy(b, _):
        pltpu.sync_copy(rows0, acc_sh.at[pl.ds(sid * RPT + b * CHUNK, CHUNK)])
        return ()
    lax.fori_loop(0, RPT // CHUNK, zcopy, ())
    pltpu.sync_copy(rows0.at[pl.ds(0, RPT % CHUNK)],
                    acc_sh.at[pl.ds(sid * RPT + (RPT // CHUNK) * CHUNK,
                                    RPT % CHUNK)])
    plsc.subcore_barrier()

    @pl.when(cid == 0)
    def _():
        pltpu.async_copy(y_hbm.at[srcv.at[0]], rows0, sem0)
        pltpu.async_copy(y_hbm.at[srcv.at[1]], rows1, sem1)

        def stage_body(s, _):
            pltpu.sync_copy(dst_hbm.at[cid, sid, pl.ds(s * SCH, SCH)], dstv)

            def body(g, _):
                j = s * SCH + 2 * g
                pltpu.make_async_copy(y_hbm.at[srcv.at[j]], rows0, sem0).wait()
                pltpu.sync_copy(rows0, acc_sh.at[dstv.at[2 * g]], add=True)

                @pl.when(j + 2 < NCH0)
                def _():
                    pltpu.async_copy(y_hbm.at[srcv.at[j + 2]], rows0, sem0)

                pltpu.make_async_copy(y_hbm.at[srcv.at[j + 1]], rows1,
                                      sem1).wait()
                pltpu.sync_copy(rows1, acc_sh.at[dstv.at[2 * g + 1]], add=True)

                @pl.when(j + 3 < NCH0)
                def _():
                    pltpu.async_copy(y_hbm.at[srcv.at[j + 3]], rows1, sem1)
                return ()
            lax.fori_loop(0, SCH // 2, body, ())
            return ()
        lax.fori_loop(0, NCH0 // SCH, stage_body, ())

    @pl.when(cid == 1)
    def _():
        def stage_body(s, _):
            pltpu.sync_copy(dst_hbm.at[cid, sid, pl.ds(s * SCH, SCH)], dstv)

            def body(g, _):
                j = s * SCH + g
                pltpu.async_copy(y_hbm.at[srcv.at[j]], rows0, sem0).wait()
                pltpu.sync_copy(rows0, acc_sh.at[dstv.at[g]], add=True)
                return ()
            lax.fori_loop(0, SCH, body, ())
            return ()
        lax.fori_loop(0, NCH1 // SCH, stage_body, ())

    plsc.subcore_barrier()
    pltpu.sync_copy(acc_sh.at[pl.ds(sid * RPT, RPT)],
                    out_hbm.at[cid, pl.ds(sid * RPT, RPT)])


_spmm_kernel = functools.partial(
    pl.kernel,
    out_type=jax.ShapeDtypeStruct((2, NACC, D), jnp.float32),
    mesh=_mesh,
    scratch_types=[
        pltpu.VMEM((NCH0, CHUNK), jnp.int32),
        pltpu.VMEM((SCH, CHUNK), jnp.int32),
        pltpu.VMEM((CHUNK, D), jnp.float32),
        pltpu.VMEM((CHUNK, D), jnp.float32),
        pltpu.VMEM_SHARED((NACC, D), jnp.float32),
        pltpu.SemaphoreType.DMA,
        pltpu.SemaphoreType.DMA,
    ],
)(_spmm_body)


BLK = 1000


def _tc_layer_body(x_ref, a0_ref, a1_ref, db_ref, wu_ref, wi_ref,
                   msg_ref, int_ref, xn_ref, yn_ref):
    i = pl.program_id(0)
    x = x_ref[...]
    db = db_ref[...]
    msg = (a0_ref[...] + a1_ref[...]) * db
    w = jnp.where(i < NU // BLK, wu_ref[...], wi_ref[...])
    logits = jnp.dot(x, w, preferred_element_type=jnp.float32)
    m = jnp.max(logits, axis=1, keepdims=True)
    e = jnp.exp(logits - m)
    p = e / jnp.sum(e, axis=1, keepdims=True)
    itl = lax.dot_general(p, w, (((1,), (1,)), ((), ())),
                          preferred_element_type=jnp.float32)
    msg_ref[...] = msg
    int_ref[...] = itl
    xn = msg + itl + x
    xn_ref[...] = xn
    yn_ref[...] = xn * db


def _tc_layer(x, a0, a1, disb, wu, wi):
    grid = (NN // BLK,)
    row_spec = pl.BlockSpec((BLK, D), lambda i: (i, 0))
    w_spec = pl.BlockSpec((D, D), lambda i: (0, 0))
    out_sds = jax.ShapeDtypeStruct((NN, D), jnp.float32)
    return pl.pallas_call(
        _tc_layer_body,
        grid=grid,
        in_specs=[row_spec, row_spec, row_spec, row_spec, w_spec, w_spec],
        out_specs=[row_spec, row_spec, row_spec, row_spec],
        out_shape=[out_sds, out_sds, out_sds, out_sds],
    )(x, a0, a1, disb, wu, wi)


def kernel(user_emb, item_emb, edge_index, user_intent, item_intent):
    h = edge_index[0].astype(jnp.int32)
    t = edge_index[1].astype(jnp.int32) + NU
    npad = EPAD - E2
    src = jnp.concatenate([jnp.stack([t, h], axis=1).reshape(-1),
                           jnp.zeros((npad,), jnp.int32)])
    dst = jnp.concatenate([jnp.stack([h, t], axis=1).reshape(-1),
                           jnp.full((npad,), NN, jnp.int32)])
    pad_tail = jnp.zeros((16, NCH0 - NCH1, CHUNK), jnp.int32)
    padd_tail = jnp.full((16, NCH0 - NCH1, CHUNK), NN, jnp.int32)
    src4 = jnp.stack([
        src[:E_SC0].reshape(16, NCH0, CHUNK),
        jnp.concatenate([src[E_SC0:].reshape(16, NCH1, CHUNK), pad_tail], 1)])
    dst4 = jnp.stack([
        dst[:E_SC0].reshape(16, NCH0, CHUNK),
        jnp.concatenate([dst[E_SC0:].reshape(16, NCH1, CHUNK), padd_tail], 1)])
    dst3 = dst4.reshape(2, 16, NCH0 * CHUNK)

    degp = _deg_kernel(dst3)
    deg = jnp.sum(degp, axis=(0, 1))[:NN]
    dis = jnp.where(deg > 0, lax.rsqrt(jnp.maximum(deg, 1.0)), 0.0)
    disb = jnp.broadcast_to(dis[:, None], (NN, D))

    e0 = jnp.concatenate([user_emb, item_emb], axis=0)
    y0 = e0 * disb

    acc0 = _spmm_kernel(y0, src4, dst4)
    msg0, int0, e1, y1 = _tc_layer(e0, acc0[0, :NN], acc0[1, :NN], disb,
                                   user_intent, item_intent)

    acc1 = _spmm_kernel(y1, src4, dst4)
    msg1, int1, e2, _ = _tc_layer(e1, acc1[0, :NN], acc1[1, :NN], disb,
                                  user_intent, item_intent)

    final = e0 + e1 + e2
    return (final[:NU], final[NU:],
            jnp.stack([msg0, msg1], axis=0),
            jnp.stack([int0, int1], axis=0))


# speedup vs baseline: 1.5040x; 1.5040x over previous
import functools

import jax
import jax.numpy as jnp
from jax import lax
from jax.experimental import pallas as pl
from jax.experimental.pallas import tpu as pltpu
from jax.experimental.pallas import tpu_sc as plsc

NU = 5000
NI = 5000
NN = NU + NI
D = 128
NACC = 10112
RPT = NACC // 16
E2 = 320000
CHUNK = 128
NCH0 = 120
NCH1 = 40
SCH = 8
EPAD = 16 * CHUNK * (NCH0 + NCH1)
E_SC0 = 16 * CHUNK * NCH0

_mesh = plsc.VectorSubcoreMesh(core_axis_name="c", subcore_axis_name="s")


def _deg_body(dst_hbm, out_hbm, idx_v, deg_v, sem):
    cid = lax.axis_index("c")
    sid = lax.axis_index("s")

    zeros16 = jnp.zeros((16,), jnp.float32)

    def zero_body(i, _):
        deg_v[pl.ds(i * 16, 16)] = zeros16
        return ()
    lax.fori_loop(0, NACC // 16, zero_body, ())

    pltpu.sync_copy(dst_hbm.at[cid, sid], idx_v)

    ones16 = jnp.ones((16,), jnp.float32)

    def body(k, _):
        idx16 = idx_v[pl.ds(k * 16, 16)]
        plsc.addupdate_scatter(deg_v, [idx16], ones16)
        return ()
    lax.fori_loop(0, jnp.where(cid == 0, NCH0 * CHUNK // 16,
                               NCH1 * CHUNK // 16), body, ())

    pltpu.sync_copy(deg_v, out_hbm.at[cid, sid])


_deg_kernel = functools.partial(
    pl.kernel,
    out_type=jax.ShapeDtypeStruct((2, 16, NACC), jnp.float32),
    mesh=_mesh,
    compiler_params=pltpu.CompilerParams(needs_layout_passes=False),
    scratch_types=[
        pltpu.VMEM((NCH0 * CHUNK,), jnp.int32),
        pltpu.VMEM((NACC,), jnp.float32),
        pltpu.SemaphoreType.DMA,
    ],
)(_deg_body)


def _spmm_body(y_hbm, src_hbm, dst_hbm, out_hbm, srcv, dstv, rows0, rows1,
               acc_sh, sem0, sem1):
    cid = lax.axis_index("c")
    sid = lax.axis_index("s")

    pltpu.sync_copy(src_hbm.at[cid, sid], srcv)

    zeros16 = jnp.zeros((16,), jnp.float32)

    def zero_body(k, _):
        r = k // (D // 16)
        c = k % (D // 16)
        rows0[r, pl.ds(c * 16, 16)] = zeros16
        return ()
    lax.fori_loop(0, CHUNK * (D // 16), zero_body, ())

    def zcopy(b, _):
        pltpu.sync_copy(rows0, acc_sh.at[pl.ds(sid * RPT + b * CHUNK, CHUNK)])
        return ()
    lax.fori_loop(0, RPT // CHUNK, zcopy, ())
    pltpu.sync_copy(rows0.at[pl.ds(0, RPT % CHUNK)],
                    acc_sh.at[pl.ds(sid * RPT + (RPT // CHUNK) * CHUNK,
                                    RPT % CHUNK)])
    plsc.subcore_barrier()

    @pl.when(cid == 0)
    def _():
        pltpu.async_copy(y_hbm.at[srcv.at[0]], rows0, sem0)
        pltpu.async_copy(y_hbm.at[srcv.at[1]], rows1, sem1)

        def stage_body(s, _):
            pltpu.sync_copy(dst_hbm.at[cid, sid, pl.ds(s * SCH, SCH)], dstv)

            def body(g, _):
                j = s * SCH + 2 * g
                pltpu.make_async_copy(y_hbm.at[srcv.at[j]], rows0, sem0).wait()
                pltpu.sync_copy(rows0, acc_sh.at[dstv.at[2 * g]], add=True)

                @pl.when(j + 2 < NCH0)
                def _():
                    pltpu.async_copy(y_hbm.at[srcv.at[j + 2]], rows0, sem0)

                pltpu.make_async_copy(y_hbm.at[srcv.at[j + 1]], rows1,
                                      sem1).wait()
                pltpu.sync_copy(rows1, acc_sh.at[dstv.at[2 * g + 1]], add=True)

                @pl.when(j + 3 < NCH0)
                def _():
                    pltpu.async_copy(y_hbm.at[srcv.at[j + 3]], rows1, sem1)
                return ()
            lax.fori_loop(0, SCH // 2, body, ())
            return ()
        lax.fori_loop(0, NCH0 // SCH, stage_body, ())

    @pl.when(cid == 1)
    def _():
        def stage_body(s, _):
            pltpu.sync_copy(dst_hbm.at[cid, sid, pl.ds(s * SCH, SCH)], dstv)

            def body(g, _):
                j = s * SCH + g
                pltpu.async_copy(y_hbm.at[srcv.at[j]], rows0, sem0).wait()
                pltpu.sync_copy(rows0, acc_sh.at[dstv.at[g]], add=True)
                return ()
            lax.fori_loop(0, SCH, body, ())
            return ()
        lax.fori_loop(0, NCH1 // SCH, stage_body, ())

    plsc.subcore_barrier()
    pltpu.sync_copy(acc_sh.at[pl.ds(sid * RPT, RPT)],
                    out_hbm.at[cid, pl.ds(sid * RPT, RPT)])


_spmm_kernel = functools.partial(
    pl.kernel,
    out_type=jax.ShapeDtypeStruct((2, NACC, D), jnp.float32),
    mesh=_mesh,
    scratch_types=[
        pltpu.VMEM((NCH0, CHUNK), jnp.int32),
        pltpu.VMEM((SCH, CHUNK), jnp.int32),
        pltpu.VMEM((CHUNK, D), jnp.float32),
        pltpu.VMEM((CHUNK, D), jnp.float32),
        pltpu.VMEM_SHARED((NACC, D), jnp.float32),
        pltpu.SemaphoreType.DMA,
        pltpu.SemaphoreType.DMA,
    ],
)(_spmm_body)


BLK = 1000


def _tc_layer_body(x_ref, a0_ref, a1_ref, db_ref, wu_ref, wi_ref,
                   msg_ref, int_ref, xn_ref, yn_ref):
    i = pl.program_id(0)
    x = x_ref[...]
    db = db_ref[...]
    msg = (a0_ref[...] + a1_ref[...]) * db
    w = jnp.where(i < NU // BLK, wu_ref[...], wi_ref[...])
    logits = jnp.dot(x, w, preferred_element_type=jnp.float32)
    m = jnp.max(logits, axis=1, keepdims=True)
    e = jnp.exp(logits - m)
    p = e / jnp.sum(e, axis=1, keepdims=True)
    itl = lax.dot_general(p, w, (((1,), (1,)), ((), ())),
                          preferred_element_type=jnp.float32)
    msg_ref[...] = msg
    int_ref[...] = itl
    xn = msg + itl + x
    xn_ref[...] = xn
    yn_ref[...] = xn * db


def _tc_layer(x, a0, a1, disb, wu, wi):
    grid = (NN // BLK,)
    row_spec = pl.BlockSpec((BLK, D), lambda i: (i, 0))
    w_spec = pl.BlockSpec((D, D), lambda i: (0, 0))
    out_sds = jax.ShapeDtypeStruct((NN, D), jnp.float32)
    return pl.pallas_call(
        _tc_layer_body,
        grid=grid,
        in_specs=[row_spec, row_spec, row_spec, row_spec, w_spec, w_spec],
        out_specs=[row_spec, row_spec, row_spec, row_spec],
        out_shape=[out_sds, out_sds, out_sds, out_sds],
    )(x, a0, a1, disb, wu, wi)


def kernel(user_emb, item_emb, edge_index, user_intent, item_intent):
    h = edge_index[0].astype(jnp.int32)
    t = edge_index[1].astype(jnp.int32) + NU
    npad = EPAD - E2
    src = jnp.concatenate([t, h, jnp.zeros((npad,), jnp.int32)])
    dst = jnp.concatenate([h, t, jnp.full((npad,), NN, jnp.int32)])
    pad_tail = jnp.zeros((16, NCH0 - NCH1, CHUNK), jnp.int32)
    padd_tail = jnp.full((16, NCH0 - NCH1, CHUNK), NN, jnp.int32)
    src4 = jnp.stack([
        src[:E_SC0].reshape(16, NCH0, CHUNK),
        jnp.concatenate([src[E_SC0:].reshape(16, NCH1, CHUNK), pad_tail], 1)])
    dst4 = jnp.stack([
        dst[:E_SC0].reshape(16, NCH0, CHUNK),
        jnp.concatenate([dst[E_SC0:].reshape(16, NCH1, CHUNK), padd_tail], 1)])
    dst3 = dst4.reshape(2, 16, NCH0 * CHUNK)

    degp = _deg_kernel(dst3)
    deg = jnp.sum(degp, axis=(0, 1))[:NN]
    dis = jnp.where(deg > 0, lax.rsqrt(jnp.maximum(deg, 1.0)), 0.0)
    disb = jnp.broadcast_to(dis[:, None], (NN, D))

    e0 = jnp.concatenate([user_emb, item_emb], axis=0)
    y0 = e0 * disb

    acc0 = _spmm_kernel(y0, src4, dst4)
    msg0, int0, e1, y1 = _tc_layer(e0, acc0[0, :NN], acc0[1, :NN], disb,
                                   user_intent, item_intent)

    acc1 = _spmm_kernel(y1, src4, dst4)
    msg1, int1, e2, _ = _tc_layer(e1, acc1[0, :NN], acc1[1, :NN], disb,
                                  user_intent, item_intent)

    final = e0 + e1 + e2
    return (final[:NU], final[NU:],
            jnp.stack([msg0, msg1], axis=0),
            jnp.stack([int0, int1], axis=0))
